# Initial kernel scaffold; baseline (speedup 1.0000x reference)
import jax
import jax.numpy as jnp
from jax.experimental import pallas as pl


def kernel(xyz, features, density, params):
    raise NotImplementedError("write your pallas kernel here")


# speedup vs baseline: 1.0008x; 1.0008x over previous
import functools

import jax
import jax.numpy as jnp
from jax import lax
from jax.experimental import pallas as pl
from jax.experimental.pallas import tpu as pltpu

_B = 4
_N = 16384
_S = 1024
_RADII = (0.5, 1.0)
_NSAMPLES = (16, 32)


def _fps_scaffold(xyz, npoint):
    b, n, _ = xyz.shape

    def body(i, state):
        idx, dists, far = state
        idx = idx.at[:, i].set(far)
        centroid = jax.vmap(lambda p, f: p[f])(xyz, far)[:, None, :]
        d = jnp.sum((xyz - centroid) ** 2, axis=-1)
        dists = jnp.minimum(dists, d)
        far = jnp.argmax(dists, axis=-1).astype(jnp.int32)
        return idx, dists, far

    idx0 = jnp.zeros((b, npoint), jnp.int32)
    d0 = jnp.full((b, n), 1e10, jnp.float32)
    f0 = jnp.zeros((b,), jnp.int32)
    idx, _, _ = jax.lax.fori_loop(0, npoint, body, (idx0, d0, f0))
    return idx


def _ball_query_scaffold(radius, nsample, xyz, new_xyz):
    n = xyz.shape[1]
    d2 = jnp.sum((new_xyz[:, :, None, :] - xyz[:, None, :, :]) ** 2, axis=-1)
    mask = d2 < radius * radius
    cand = jnp.where(mask, jnp.arange(n)[None, None, :], n)
    negv, _ = jax.lax.top_k(-cand, nsample)
    idx = -negv
    first = idx[..., :1]
    idx = jnp.where(idx == n, jnp.where(first == n, 0, first), idx)
    return idx.astype(jnp.int32)


def _bn_from_stats(stats, gamma, beta, count):
    mean = stats[:, 0:1] / count
    var = stats[:, 1:2] / count - mean * mean
    scale = gamma[:, None] / jnp.sqrt(var + 1e-5)
    shift = beta[:, None] - mean * scale
    return scale, shift


def _bn_relu_inline(y, gamma, beta):
    mean = jnp.mean(y, axis=1, keepdims=True)
    var = jnp.mean((y - mean) ** 2, axis=1, keepdims=True)
    y = (y - mean) / jnp.sqrt(var + 1e-5) * gamma[:, None] + beta[:, None]
    return jax.nn.relu(y)


def _layer_body(count, x_ref, w_ref, stats_in_ref, gamma_ref, beta_ref,
                y_ref, stats_ref):
    x = x_ref[...]
    if stats_in_ref is not None:
        scale, shift = _bn_from_stats(stats_in_ref[...], gamma_ref[...],
                                      beta_ref[...], count)
        x = jax.nn.relu(x * scale + shift)
    y = jnp.dot(w_ref[...], x, preferred_element_type=jnp.float32)
    y_ref[...] = y
    s1 = jnp.sum(y, axis=1, keepdims=True)
    s2 = jnp.sum(y * y, axis=1, keepdims=True)
    part = jnp.concatenate([s1, s2], axis=1)

    @pl.when(pl.program_id(0) == 0)
    def _init():
        stats_ref[...] = jnp.zeros_like(stats_ref)

    stats_ref[...] += part


def _layer_pass(x, w, stats_in, gamma, beta, chunk):
    cin, l = x.shape
    cout = w.shape[0]
    nsteps = l // chunk
    grid = (nsteps,)
    in_specs = [
        pl.BlockSpec((cin, chunk), lambda i: (0, i)),
        pl.BlockSpec((cout, cin), lambda i: (0, 0)),
    ]
    args = [x, w]
    if stats_in is not None:
        in_specs += [
            pl.BlockSpec((cin, 2), lambda i: (0, 0)),
            pl.BlockSpec((cin,), lambda i: (0,)),
            pl.BlockSpec((cin,), lambda i: (0,)),
        ]
        args += [stats_in, gamma, beta]
        body = functools.partial(_layer_body, float(l))
    else:
        body = lambda x_ref, w_ref, y_ref, stats_ref: _layer_body(
            float(l), x_ref, w_ref, None, None, None, y_ref, stats_ref)
    out_specs = (
        pl.BlockSpec((cout, chunk), lambda i: (0, i)),
        pl.BlockSpec((cout, 2), lambda i: (0, 0)),
    )
    return pl.pallas_call(
        body,
        grid=grid,
        in_specs=in_specs,
        out_specs=out_specs,
        out_shape=(jax.ShapeDtypeStruct((cout, l), jnp.float32),
                   jax.ShapeDtypeStruct((cout, 2), jnp.float32)),
    )(*args)


def _pool_body(count, ns, y_ref, stats_ref, gamma_ref, beta_ref, out_ref):
    scale, shift = _bn_from_stats(stats_ref[...], gamma_ref[...],
                                  beta_ref[...], count)
    x = jax.nn.relu(y_ref[...] * scale + shift)
    c, l = x.shape
    x = x.reshape(c, l // ns, ns)
    out_ref[...] = jnp.max(x, axis=2)


def _pool_pass(y, stats, gamma, beta, ns, chunk):
    c, l = y.shape
    bs = l // ns
    nsteps = bs // chunk
    return pl.pallas_call(
        functools.partial(_pool_body, float(l), ns),
        grid=(nsteps,),
        in_specs=[
            pl.BlockSpec((c, chunk * ns), lambda i: (0, i)),
            pl.BlockSpec((c, 2), lambda i: (0, 0)),
            pl.BlockSpec((c,), lambda i: (0,)),
            pl.BlockSpec((c,), lambda i: (0,)),
        ],
        out_specs=pl.BlockSpec((c, chunk), lambda i: (0, i)),
        out_shape=jax.ShapeDtypeStruct((c, bs), jnp.float32),
    )(y, stats, gamma, beta)


def _head_body(w_agg, g_agg, b_agg, w_conf, g_conf, b_conf, w_out, b_out,
               p0_ref, p1_ref, feats_ref, scores_ref):
    feats = jnp.concatenate([p0_ref[...], p1_ref[...]], axis=0)
    feats = _bn_relu_inline(
        jnp.dot(w_agg[...], feats, preferred_element_type=jnp.float32),
        g_agg[...], b_agg[...])
    feats_ref[...] = feats
    h = _bn_relu_inline(
        jnp.dot(w_conf[...], feats, preferred_element_type=jnp.float32),
        g_conf[...], b_conf[...])
    scores = (jnp.dot(w_out[...], h, preferred_element_type=jnp.float32)
              + b_out[...][:, None])
    scores_ref[...] = scores


def _dense_tail(g0, g1, params):
    pooled = []
    for si, (g, ns) in enumerate(((g0, _NSAMPLES[0]), (g1, _NSAMPLES[1]))):
        layers = params['mlp%d' % si]
        chunk = 16384
        y, stats = _layer_pass(g, layers[0]['W'], None, None, None, chunk)
        for li in (1, 2):
            y, stats = _layer_pass(y, layers[li]['W'], stats,
                                   layers[li - 1]['gamma'],
                                   layers[li - 1]['beta'], chunk)
        pooled.append(_pool_pass(y, stats, layers[2]['gamma'],
                                 layers[2]['beta'], ns, 1024))
    pa, pc, po = params['agg'], params['conf'], params['conf_out']
    return pl.pallas_call(
        functools.partial(_head_body),
        out_shape=(jax.ShapeDtypeStruct((64, _B * _S), jnp.float32),
                   jax.ShapeDtypeStruct((1, _B * _S), jnp.float32)),
    )(pa['W'], pa['gamma'], pa['beta'], pc['W'], pc['gamma'], pc['beta'],
      po['W'], po['b'], pooled[0], pooled[1])


def kernel(xyz, features, density, params):
    sample_idx = _fps_scaffold(xyz, _S)
    new_xyz = jax.vmap(lambda p, i: p[i])(xyz, sample_idx)
    new_density = jax.vmap(lambda d, i: d[:, i])(density, sample_idx)

    gs = []
    for si, (radius, nsample) in enumerate(zip(_RADII, _NSAMPLES)):
        idx = _ball_query_scaffold(radius, nsample, xyz, new_xyz)
        grouped_xyz = jax.vmap(lambda p, i: p[i])(xyz, idx) - new_xyz[:, :, None, :]
        grouped_feat = jax.vmap(lambda f, i: f[:, i])(features, idx)
        nf = jnp.concatenate(
            [jnp.transpose(grouped_xyz, (0, 3, 1, 2)), grouped_feat], axis=1)
        gs.append(jnp.transpose(nf, (1, 0, 2, 3)).reshape(4, -1))

    feats, scores = _dense_tail(gs[0], gs[1], params)
    new_features = jnp.transpose(feats.reshape(64, _B, _S), (1, 0, 2))
    new_scores = scores.reshape(_B, _S)
    return (new_xyz, new_features, new_density, new_scores)


# speedup vs baseline: 1.3805x; 1.3794x over previous
import functools

import jax
import jax.numpy as jnp
from jax import lax
from jax.experimental import pallas as pl
from jax.experimental.pallas import tpu as pltpu
from jax.experimental.pallas import tpu_sc as plsc

_B = 4
_N = 16384
_S = 1024
_RADII = (0.5, 1.0)
_NSAMPLES = (16, 32)


_SLAB = _N // 8
_LANES = 16


def _fps_sc_body(x_hbm, y_hbm, z_hbm, d_hbm, idx_hbm, nx_hbm, ny_hbm,
                 nz_hbm, nd_hbm, xf, yf, zf, df, dists, out_idx, gbuf,
                 shared, red, nbuf):
    c = lax.axis_index("c")
    s = lax.axis_index("s")
    b = c * 2 + s // 8
    slab = s % 8
    slab_off = slab * _SLAB
    lanes = lax.iota(jnp.int32, _LANES)
    lane0 = lanes == 0

    pltpu.sync_copy(x_hbm.at[b], xf)
    pltpu.sync_copy(y_hbm.at[b], yf)
    pltpu.sync_copy(z_hbm.at[b], zf)
    pltpu.sync_copy(d_hbm.at[b], df)

    big = jnp.full((_LANES,), 1e10, jnp.float32)
    for j in range(_SLAB // _LANES):
        dists[pl.ds(j * _LANES, _LANES)] = big

    def round_body(i, far):
        plsc.store_scatter(out_idx, [jnp.full((_LANES,), i, jnp.int32)],
                           far, mask=lane0)
        cx = plsc.load_gather(xf, [far])
        cy = plsc.load_gather(yf, [far])
        cz = plsc.load_gather(zf, [far])

        def scan_body(j, carry):
            vmax, vidx = carry
            base = j * _LANES
            dx = xf[pl.ds(slab_off + base, _LANES)] - cx
            dy = yf[pl.ds(slab_off + base, _LANES)] - cy
            dz = zf[pl.ds(slab_off + base, _LANES)] - cz
            d = dx * dx + dy * dy + dz * dz
            nd = jnp.minimum(dists[pl.ds(base, _LANES)], d)
            dists[pl.ds(base, _LANES)] = nd
            better = nd > vmax
            vmax = jnp.where(better, nd, vmax)
            vidx = jnp.where(better, slab_off + base + lanes, vidx)
            return vmax, vidx

        vmax0 = jnp.full((_LANES,), -1.0, jnp.float32)
        vidx0 = jnp.zeros((_LANES,), jnp.int32)
        vmax, vidx = lax.fori_loop(0, _SLAB // _LANES, scan_body,
                                   (vmax0, vidx0))
        m = jnp.max(vmax)
        cand = jnp.where(vmax == m, vidx, _N)
        lidx = jnp.min(cand)
        par = i % 2
        gbuf[pl.ds(0, _LANES)] = plsc.bitcast(
            jnp.full((_LANES,), m, jnp.float32), jnp.int32)
        gbuf[pl.ds(_LANES, _LANES)] = jnp.full((_LANES,), lidx, jnp.int32)
        pltpu.sync_copy(gbuf, shared.at[pl.ds(par * 512 + s * 32, 32)])
        plsc.subcore_barrier()
        group = (s // 8) * 8
        pltpu.sync_copy(shared.at[pl.ds(par * 512 + group * 32, 256)], red)

        bmax = jnp.full((_LANES,), -1.0, jnp.float32)
        bidx = jnp.zeros((_LANES,), jnp.int32)
        for k in range(8):
            v = plsc.bitcast(red[pl.ds(k * 32, _LANES)], jnp.float32)
            ix = red[pl.ds(k * 32 + _LANES, _LANES)]
            better = v > bmax
            bmax = jnp.where(better, v, bmax)
            bidx = jnp.where(better, ix, bidx)
        return jnp.where((bidx >= 0) & (bidx < _N), bidx, 0)

    far0 = jnp.zeros((_LANES,), jnp.int32)
    lax.fori_loop(0, _S, round_body, far0)

    @pl.when(slab == 0)
    def _write():
        for t in range(_S // _LANES):
            iv = out_idx[pl.ds(t * _LANES, _LANES)]
            nbuf[0, pl.ds(t * _LANES, _LANES)] = plsc.load_gather(xf, [iv])
            nbuf[1, pl.ds(t * _LANES, _LANES)] = plsc.load_gather(yf, [iv])
            nbuf[2, pl.ds(t * _LANES, _LANES)] = plsc.load_gather(zf, [iv])
            nbuf[3, pl.ds(t * _LANES, _LANES)] = plsc.load_gather(df, [iv])
        pltpu.sync_copy(nbuf.at[0], nx_hbm.at[b])
        pltpu.sync_copy(nbuf.at[1], ny_hbm.at[b])
        pltpu.sync_copy(nbuf.at[2], nz_hbm.at[b])
        pltpu.sync_copy(nbuf.at[3], nd_hbm.at[b])
        pltpu.sync_copy(out_idx, idx_hbm.at[b])


def _fps_sc(xs, ys, zs, den):
    mesh = plsc.VectorSubcoreMesh(core_axis_name="c", subcore_axis_name="s")
    f = pl.kernel(
        _fps_sc_body,
        out_type=[
            jax.ShapeDtypeStruct((_B, _S), jnp.int32),
            jax.ShapeDtypeStruct((_B, _S), jnp.float32),
            jax.ShapeDtypeStruct((_B, _S), jnp.float32),
            jax.ShapeDtypeStruct((_B, _S), jnp.float32),
            jax.ShapeDtypeStruct((_B, _S), jnp.float32),
        ],
        mesh=mesh,
        scratch_types=[
            pltpu.VMEM((_N,), jnp.float32),
            pltpu.VMEM((_N,), jnp.float32),
            pltpu.VMEM((_N,), jnp.float32),
            pltpu.VMEM((_N,), jnp.float32),
            pltpu.VMEM((_SLAB,), jnp.float32),
            pltpu.VMEM((_S,), jnp.int32),
            pltpu.VMEM((32,), jnp.int32),
            pltpu.VMEM_SHARED((1024,), jnp.int32),
            pltpu.VMEM((256,), jnp.int32),
            pltpu.VMEM((4, _S), jnp.float32),
        ],
        compiler_params=pltpu.CompilerParams(needs_layout_passes=False),
    )
    return f(xs, ys, zs, den)


def _make_bq_body(r2, ns, early_exit):
    cap = 160
    kch = 8
    nchunks = _N // (kch * _LANES)

    def body(x_hbm, y_hbm, z_hbm, f_hbm, si_hbm, g_hbm, xf, yf, zf, ff,
             oidx, idxbuf, obuf):
        c = lax.axis_index("c")
        s = lax.axis_index("s")
        b = c * 2 + s // 8
        slab = s % 8
        lanes = lax.iota(jnp.int32, _LANES)
        zero16 = jnp.zeros((_LANES,), jnp.int32)

        pltpu.sync_copy(x_hbm.at[b], xf)
        pltpu.sync_copy(y_hbm.at[b], yf)
        pltpu.sync_copy(z_hbm.at[b], zf)
        pltpu.sync_copy(f_hbm.at[b], ff)
        pltpu.sync_copy(si_hbm.at[b], oidx)

        def do_query(qi, _):
            q = slab * 128 + qi
            iv = plsc.load_gather(oidx,
                                  [jnp.full((_LANES,), q, jnp.int32)])
            cx = plsc.load_gather(xf, [iv])
            cy = plsc.load_gather(yf, [iv])
            cz = plsc.load_gather(zf, [iv])

            def slices(base, cnt_v):
                for k in range(kch):
                    off = base + k * _LANES
                    dx = xf[pl.ds(off, _LANES)] - cx
                    dy = yf[pl.ds(off, _LANES)] - cy
                    dz = zf[pl.ds(off, _LANES)] - cz
                    d2 = dx * dx + dy * dy + dz * dz
                    mv = d2 < r2
                    cs = plsc.cumsum(jnp.where(mv, 1, 0))
                    pos = cnt_v + cs - 1
                    plsc.store_scatter(idxbuf, [pos], off + lanes,
                                       mask=mv & (pos < cap))
                    cnt_v = cnt_v + plsc.all_reduce_population_count(mv)
                return cnt_v

            if early_exit:
                def cond(st):
                    j, cnt_s = st[0], st[1]
                    return (j < nchunks) & (cnt_s < ns)

                def chunk(st):
                    j, _, cnt_v = st
                    cnt_v = slices(j * (kch * _LANES), cnt_v)
                    return j + 1, jnp.max(cnt_v), cnt_v

                _, _, cnt_v = lax.while_loop(
                    cond, chunk, (jnp.int32(0), jnp.int32(0), zero16))
            else:
                cnt_v = lax.fori_loop(
                    0, nchunks,
                    lambda j, cv: slices(j * (kch * _LANES), cv), zero16)

            first = plsc.load_gather(idxbuf, [zero16])
            for t in range(ns // _LANES):
                cur = idxbuf[pl.ds(t * _LANES, _LANES)]
                sel = jnp.where(t * _LANES + lanes < cnt_v, cur, first)
                gx = plsc.load_gather(xf, [sel]) - cx
                gy = plsc.load_gather(yf, [sel]) - cy
                gz = plsc.load_gather(zf, [sel]) - cz
                gf = plsc.load_gather(ff, [sel])
                o = qi * ns + t * _LANES
                obuf[pl.ds(o, _LANES)] = gx
                obuf[pl.ds(128 * ns + o, _LANES)] = gy
                obuf[pl.ds(2 * 128 * ns + o, _LANES)] = gz
                obuf[pl.ds(3 * 128 * ns + o, _LANES)] = gf
            return 0

        lax.fori_loop(0, 128, do_query, 0)
        w = b * 8 + slab
        pltpu.sync_copy(obuf,
                        g_hbm.at[pl.ds(w * 4 * 128 * ns, 4 * 128 * ns)])

    return body


def _bq_sc(r2, ns, early_exit, xs, ys, zs, feat, sample_idx):
    mesh = plsc.VectorSubcoreMesh(core_axis_name="c", subcore_axis_name="s")
    f = pl.kernel(
        _make_bq_body(r2, ns, early_exit),
        out_type=[jax.ShapeDtypeStruct((_B * _S * 4 * ns,), jnp.float32)],
        mesh=mesh,
        scratch_types=[
            pltpu.VMEM((_N,), jnp.float32),
            pltpu.VMEM((_N,), jnp.float32),
            pltpu.VMEM((_N,), jnp.float32),
            pltpu.VMEM((_N,), jnp.float32),
            pltpu.VMEM((_S,), jnp.int32),
            pltpu.VMEM((160,), jnp.int32),
            pltpu.VMEM((4 * 128 * ns,), jnp.float32),
        ],
        compiler_params=pltpu.CompilerParams(needs_layout_passes=False),
    )
    return f(xs, ys, zs, feat, sample_idx)[0]


def _bn_from_stats(stats, gamma, beta, count):
    mean = stats[:, 0:1] / count
    var = stats[:, 1:2] / count - mean * mean
    scale = gamma[:, None] / jnp.sqrt(var + 1e-5)
    shift = beta[:, None] - mean * scale
    return scale, shift


def _bn_relu_inline(y, gamma, beta):
    mean = jnp.mean(y, axis=1, keepdims=True)
    var = jnp.mean((y - mean) ** 2, axis=1, keepdims=True)
    y = (y - mean) / jnp.sqrt(var + 1e-5) * gamma[:, None] + beta[:, None]
    return jax.nn.relu(y)


def _layer_body(count, x_ref, w_ref, stats_in_ref, gamma_ref, beta_ref,
                y_ref, stats_ref):
    x = x_ref[...]
    if stats_in_ref is not None:
        scale, shift = _bn_from_stats(stats_in_ref[...], gamma_ref[...],
                                      beta_ref[...], count)
        x = jax.nn.relu(x * scale + shift)
    y = jnp.dot(w_ref[...], x, preferred_element_type=jnp.float32)
    y_ref[...] = y
    s1 = jnp.sum(y, axis=1, keepdims=True)
    s2 = jnp.sum(y * y, axis=1, keepdims=True)
    part = jnp.concatenate([s1, s2], axis=1)

    @pl.when(pl.program_id(0) == 0)
    def _init():
        stats_ref[...] = jnp.zeros_like(stats_ref)

    stats_ref[...] += part


def _layer_pass(x, w, stats_in, gamma, beta, chunk):
    cin, l = x.shape
    cout = w.shape[0]
    nsteps = l // chunk
    grid = (nsteps,)
    in_specs = [
        pl.BlockSpec((cin, chunk), lambda i: (0, i)),
        pl.BlockSpec((cout, cin), lambda i: (0, 0)),
    ]
    args = [x, w]
    if stats_in is not None:
        in_specs += [
            pl.BlockSpec((cin, 2), lambda i: (0, 0)),
            pl.BlockSpec((cin,), lambda i: (0,)),
            pl.BlockSpec((cin,), lambda i: (0,)),
        ]
        args += [stats_in, gamma, beta]
        body = functools.partial(_layer_body, float(l))
    else:
        body = lambda x_ref, w_ref, y_ref, stats_ref: _layer_body(
            float(l), x_ref, w_ref, None, None, None, y_ref, stats_ref)
    out_specs = (
        pl.BlockSpec((cout, chunk), lambda i: (0, i)),
        pl.BlockSpec((cout, 2), lambda i: (0, 0)),
    )
    return pl.pallas_call(
        body,
        grid=grid,
        in_specs=in_specs,
        out_specs=out_specs,
        out_shape=(jax.ShapeDtypeStruct((cout, l), jnp.float32),
                   jax.ShapeDtypeStruct((cout, 2), jnp.float32)),
    )(*args)


def _pool_body(count, ns, y_ref, stats_ref, gamma_ref, beta_ref, out_ref):
    scale, shift = _bn_from_stats(stats_ref[...], gamma_ref[...],
                                  beta_ref[...], count)
    x = jax.nn.relu(y_ref[...] * scale + shift)
    c, l = x.shape
    x = x.reshape(c, l // ns, ns)
    out_ref[...] = jnp.max(x, axis=2)


def _pool_pass(y, stats, gamma, beta, ns, chunk):
    c, l = y.shape
    bs = l // ns
    nsteps = bs // chunk
    return pl.pallas_call(
        functools.partial(_pool_body, float(l), ns),
        grid=(nsteps,),
        in_specs=[
            pl.BlockSpec((c, chunk * ns), lambda i: (0, i)),
            pl.BlockSpec((c, 2), lambda i: (0, 0)),
            pl.BlockSpec((c,), lambda i: (0,)),
            pl.BlockSpec((c,), lambda i: (0,)),
        ],
        out_specs=pl.BlockSpec((c, chunk), lambda i: (0, i)),
        out_shape=jax.ShapeDtypeStruct((c, bs), jnp.float32),
    )(y, stats, gamma, beta)


def _head_body(w_agg, g_agg, b_agg, w_conf, g_conf, b_conf, w_out, b_out,
               p0_ref, p1_ref, feats_ref, scores_ref):
    feats = jnp.concatenate([p0_ref[...], p1_ref[...]], axis=0)
    feats = _bn_relu_inline(
        jnp.dot(w_agg[...], feats, preferred_element_type=jnp.float32),
        g_agg[...], b_agg[...])
    feats_ref[...] = feats
    h = _bn_relu_inline(
        jnp.dot(w_conf[...], feats, preferred_element_type=jnp.float32),
        g_conf[...], b_conf[...])
    scores = (jnp.dot(w_out[...], h, preferred_element_type=jnp.float32)
              + b_out[...][:, None])
    scores_ref[...] = scores


def _dense_tail(g0, g1, params):
    pooled = []
    for si, (g, ns) in enumerate(((g0, _NSAMPLES[0]), (g1, _NSAMPLES[1]))):
        layers = params['mlp%d' % si]
        chunk = 16384
        y, stats = _layer_pass(g, layers[0]['W'], None, None, None, chunk)
        for li in (1, 2):
            y, stats = _layer_pass(y, layers[li]['W'], stats,
                                   layers[li - 1]['gamma'],
                                   layers[li - 1]['beta'], chunk)
        pooled.append(_pool_pass(y, stats, layers[2]['gamma'],
                                 layers[2]['beta'], ns, 1024))
    pa, pc, po = params['agg'], params['conf'], params['conf_out']
    return pl.pallas_call(
        functools.partial(_head_body),
        out_shape=(jax.ShapeDtypeStruct((64, _B * _S), jnp.float32),
                   jax.ShapeDtypeStruct((1, _B * _S), jnp.float32)),
    )(pa['W'], pa['gamma'], pa['beta'], pc['W'], pc['gamma'], pc['beta'],
      po['W'], po['b'], pooled[0], pooled[1])


def kernel(xyz, features, density, params):
    xs = xyz[:, :, 0]
    ys = xyz[:, :, 1]
    zs = xyz[:, :, 2]
    den = density[:, 0, :]
    feat = features[:, 0, :]
    sample_idx, nx, ny, nz, nd = _fps_sc(xs, ys, zs, den)
    new_xyz = jnp.stack([nx, ny, nz], axis=-1)
    new_density = nd[:, None, :]

    n = _N
    d2 = jnp.sum((new_xyz[:, :, None, :] - xyz[:, None, :, :]) ** 2,
                 axis=-1)
    gs = []
    for radius, nsample in zip(_RADII, _NSAMPLES):
        mask = d2 < radius * radius
        cand = jnp.where(mask, jnp.arange(n)[None, None, :], n)
        negv, _ = jax.lax.top_k(-cand, nsample)
        idx = -negv
        first = idx[..., :1]
        idx = jnp.where(idx == n, jnp.where(first == n, 0, first),
                        idx).astype(jnp.int32)
        grouped_xyz = (jax.vmap(lambda p, i: p[i])(xyz, idx)
                       - new_xyz[:, :, None, :])
        grouped_feat = jax.vmap(lambda f, i: f[:, i])(features, idx)
        nf = jnp.concatenate(
            [jnp.transpose(grouped_xyz, (0, 3, 1, 2)), grouped_feat],
            axis=1)
        gs.append(jnp.transpose(nf, (1, 0, 2, 3)).reshape(4, -1))

    feats, scores = _dense_tail(gs[0], gs[1], params)
    new_features = jnp.transpose(feats.reshape(64, _B, _S), (1, 0, 2))
    new_scores = scores.reshape(_B, _S)
    return (new_xyz, new_features, new_density, new_scores)
